# baseline (speedup 1.0000x reference)
import functools

import jax
import jax.numpy as jnp
from jax import lax
from jax.experimental import pallas as pl
from jax.experimental.pallas import tpu as pltpu
from jax.experimental.pallas import tpu_sc as plsc

BATCH = 16384
EMBED_DIM = 64
ROW_PAD = 128

_NUM_WORKERS = 32
_B_PER_W = BATCH // _NUM_WORKERS
_CHUNK = 128
_NCHUNK = _B_PER_W // _CHUNK
_LANES = 16


def _gather_kernel(idx_hbm, table_hbm, out_hbm, idx_v, g0, g1, s0, s1,
                   gs0, gs1, ws0, ws1):
    wid = lax.axis_index("s") * 2 + lax.axis_index("c")
    base = wid * _B_PER_W
    pltpu.sync_copy(idx_hbm.at[pl.ds(base, _B_PER_W)], idx_v)
    g, s, gs, ws = [g0, g1], [s0, s1], [gs0, gs1], [ws0, ws1]

    def fire_gather(j):
        return pltpu.async_copy(
            table_hbm.at[idx_v.at[pl.ds(j * _CHUNK, _CHUNK)]],
            g[j % 2],
            gs[j % 2],
        )

    gathers = {0: fire_gather(0), 1: fire_gather(1)}
    writes = {}
    for j in range(_NCHUNK):
        gathers[j].wait()
        if j >= 2:
            writes[j - 2].wait()
        gbuf, sbuf = g[j % 2], s[j % 2]

        def body(r, carry, gbuf=gbuf, sbuf=sbuf):
            for k in range(EMBED_DIM // _LANES):
                sbuf[r, pl.ds(k * _LANES, _LANES)] = (
                    gbuf[r, pl.ds(k * _LANES, _LANES)]
                )
            return carry

        lax.fori_loop(0, _CHUNK, body, 0)
        writes[j] = pltpu.async_copy(
            sbuf,
            out_hbm.at[pl.ds(base + j * _CHUNK, _CHUNK)],
            ws[j % 2],
        )
        if j + 2 < _NCHUNK:
            gathers[j + 2] = fire_gather(j + 2)
    writes[_NCHUNK - 2].wait()
    writes[_NCHUNK - 1].wait()


@jax.jit
def _lookup(item_ids, embedding_table):
    mesh = plsc.VectorSubcoreMesh(core_axis_name="c", subcore_axis_name="s")
    table_pad = jnp.pad(embedding_table, ((0, 0), (0, ROW_PAD - EMBED_DIM)))
    kern = functools.partial(
        pl.kernel,
        mesh=mesh,
        out_type=jax.ShapeDtypeStruct((BATCH, EMBED_DIM), jnp.float32),
        scratch_types=[
            pltpu.VMEM((_B_PER_W,), jnp.int32),
            pltpu.VMEM((_CHUNK, ROW_PAD), jnp.float32),
            pltpu.VMEM((_CHUNK, ROW_PAD), jnp.float32),
            pltpu.VMEM((_CHUNK, EMBED_DIM), jnp.float32),
            pltpu.VMEM((_CHUNK, EMBED_DIM), jnp.float32),
            pltpu.SemaphoreType.DMA,
            pltpu.SemaphoreType.DMA,
            pltpu.SemaphoreType.DMA,
            pltpu.SemaphoreType.DMA,
        ],
    )(_gather_kernel)
    return kern(item_ids, table_pad)


def kernel(item_ids, embedding_table):
    return _lookup(item_ids.astype(jnp.int32), embedding_table)


# speedup vs baseline: 1.1892x; 1.1892x over previous
import functools

import jax
import jax.numpy as jnp
from jax import lax
from jax.experimental import pallas as pl
from jax.experimental.pallas import tpu as pltpu
from jax.experimental.pallas import tpu_sc as plsc

BATCH = 16384
EMBED_DIM = 64
ROW_PAD = 128

_NUM_WORKERS = 32
_B_PER_W = BATCH // _NUM_WORKERS
_CHUNK = 128
_NCHUNK = _B_PER_W // _CHUNK
_LANES = 16


def _gather_kernel(idx_hbm, table_hbm, out_hbm, idx_v, g0, g1, g2, g3,
                   t0, t1, gs0, gs1, gs2, gs3, ws0, ws1):
    wid = lax.axis_index("s") * 2 + lax.axis_index("c")
    base = wid * _B_PER_W
    pltpu.sync_copy(idx_hbm.at[pl.ds(base, _B_PER_W)], idx_v)
    g, t = [g0, g1, g2, g3], [t0, t1]
    gs, ws = [gs0, gs1, gs2, gs3], [ws0, ws1]
    gathers = [
        pltpu.async_copy(
            table_hbm.at[idx_v.at[pl.ds(j * _CHUNK, _CHUNK)]],
            g[j],
            gs[j],
        )
        for j in range(_NCHUNK)
    ]

    iota = lax.iota(jnp.int32, _LANES)
    diags = [jnp.bitwise_and(iota + d, _LANES - 1) for d in range(_LANES)]
    _EBLK = EMBED_DIM // _LANES
    _BBLK = _CHUNK // _LANES
    writes = {}
    for j in range(_NCHUNK):
        gathers[j].wait()
        if j >= 2:
            writes[j - 2].wait()
        gbuf, tbuf = g[j], t[j % 2]

        @plsc.parallel_loop(0, _EBLK * _BBLK, unroll=2)
        def _(blk, gbuf=gbuf, tbuf=tbuf):
            e0 = (blk % _EBLK) * _LANES
            b0 = (blk // _EBLK) * _LANES
            rowv = iota + b0
            for d in range(_LANES):
                colv = diags[d] + e0
                vals = plsc.load_gather(gbuf, [rowv, colv])
                plsc.store_scatter(tbuf, [colv, rowv], vals)

        writes[j] = pltpu.async_copy(
            tbuf,
            out_hbm.at[:, pl.ds(base + j * _CHUNK, _CHUNK)],
            ws[j % 2],
        )
    writes[_NCHUNK - 2].wait()
    writes[_NCHUNK - 1].wait()


@jax.jit
def _lookup(item_ids, embedding_table):
    mesh = plsc.VectorSubcoreMesh(core_axis_name="c", subcore_axis_name="s")
    table_pad = jnp.pad(embedding_table, ((0, 0), (0, ROW_PAD - EMBED_DIM)))
    kern = functools.partial(
        pl.kernel,
        mesh=mesh,
        compiler_params=pltpu.CompilerParams(needs_layout_passes=False),
        out_type=jax.ShapeDtypeStruct((EMBED_DIM, BATCH), jnp.float32),
        scratch_types=[
            pltpu.VMEM((_B_PER_W,), jnp.int32),
            pltpu.VMEM((_CHUNK, ROW_PAD), jnp.float32),
            pltpu.VMEM((_CHUNK, ROW_PAD), jnp.float32),
            pltpu.VMEM((_CHUNK, ROW_PAD), jnp.float32),
            pltpu.VMEM((_CHUNK, ROW_PAD), jnp.float32),
            pltpu.VMEM((EMBED_DIM, _CHUNK), jnp.float32),
            pltpu.VMEM((EMBED_DIM, _CHUNK), jnp.float32),
            pltpu.SemaphoreType.DMA,
            pltpu.SemaphoreType.DMA,
            pltpu.SemaphoreType.DMA,
            pltpu.SemaphoreType.DMA,
            pltpu.SemaphoreType.DMA,
            pltpu.SemaphoreType.DMA,
        ],
    )(_gather_kernel)
    return jnp.transpose(kern(item_ids, table_pad))


def kernel(item_ids, embedding_table):
    return _lookup(item_ids.astype(jnp.int32), embedding_table)


# speedup vs baseline: 1.3221x; 1.1117x over previous
import functools

import jax
import jax.numpy as jnp
from jax import lax
from jax.experimental import pallas as pl
from jax.experimental.pallas import tpu as pltpu
from jax.experimental.pallas import tpu_sc as plsc

BATCH = 16384
EMBED_DIM = 64
ROW_PAD = 128

_NUM_WORKERS = 32
_B_PER_W = BATCH // _NUM_WORKERS
_CHUNK = 128
_NCHUNK = _B_PER_W // _CHUNK
_LANES = 16


def _gather_kernel(idx_hbm, table_hbm, out_hbm, idx_v, table_s, g0, g1, g2,
                   g3, t0, t1, gs0, gs1, gs2, gs3, ws0, ws1):
    sid = lax.axis_index("s")
    wid = sid * 2 + lax.axis_index("c")
    base = wid * _B_PER_W

    @pl.when(sid == 0)
    def _():
        pltpu.sync_copy(table_hbm, table_s)

    pltpu.sync_copy(idx_hbm.at[pl.ds(base, _B_PER_W)], idx_v)
    plsc.subcore_barrier()
    g, t = [g0, g1, g2, g3], [t0, t1]
    gs, ws = [gs0, gs1, gs2, gs3], [ws0, ws1]
    gathers = [
        pltpu.async_copy(
            table_s.at[idx_v.at[pl.ds(j * _CHUNK, _CHUNK)]],
            g[j],
            gs[j],
        )
        for j in range(_NCHUNK)
    ]

    iota = lax.iota(jnp.int32, _LANES)
    diags = [jnp.bitwise_and(iota + d, _LANES - 1) for d in range(_LANES)]
    _EBLK = EMBED_DIM // _LANES
    _BBLK = _CHUNK // _LANES
    writes = {}
    for j in range(_NCHUNK):
        gathers[j].wait()
        if j >= 2:
            writes[j - 2].wait()
        gbuf, tbuf = g[j], t[j % 2]

        @plsc.parallel_loop(0, _EBLK * _BBLK, unroll=2)
        def _(blk, gbuf=gbuf, tbuf=tbuf):
            e0 = (blk % _EBLK) * _LANES
            b0 = (blk // _EBLK) * _LANES
            rowv = iota + b0
            for d in range(_LANES):
                colv = diags[d] + e0
                vals = plsc.load_gather(gbuf, [rowv, colv])
                plsc.store_scatter(tbuf, [colv, rowv], vals)

        writes[j] = pltpu.async_copy(
            tbuf,
            out_hbm.at[:, pl.ds(base + j * _CHUNK, _CHUNK)],
            ws[j % 2],
        )
    writes[_NCHUNK - 2].wait()
    writes[_NCHUNK - 1].wait()


@jax.jit
def _lookup(item_ids, embedding_table):
    mesh = plsc.VectorSubcoreMesh(core_axis_name="c", subcore_axis_name="s")
    table_pad = jnp.pad(embedding_table, ((0, 0), (0, ROW_PAD - EMBED_DIM)))
    kern = functools.partial(
        pl.kernel,
        mesh=mesh,
        compiler_params=pltpu.CompilerParams(needs_layout_passes=False),
        out_type=jax.ShapeDtypeStruct((EMBED_DIM, BATCH), jnp.float32),
        scratch_types=[
            pltpu.VMEM((_B_PER_W,), jnp.int32),
            pltpu.VMEM_SHARED((1001, ROW_PAD), jnp.float32),
            pltpu.VMEM((_CHUNK, ROW_PAD), jnp.float32),
            pltpu.VMEM((_CHUNK, ROW_PAD), jnp.float32),
            pltpu.VMEM((_CHUNK, ROW_PAD), jnp.float32),
            pltpu.VMEM((_CHUNK, ROW_PAD), jnp.float32),
            pltpu.VMEM((EMBED_DIM, _CHUNK), jnp.float32),
            pltpu.VMEM((EMBED_DIM, _CHUNK), jnp.float32),
            pltpu.SemaphoreType.DMA,
            pltpu.SemaphoreType.DMA,
            pltpu.SemaphoreType.DMA,
            pltpu.SemaphoreType.DMA,
            pltpu.SemaphoreType.DMA,
            pltpu.SemaphoreType.DMA,
        ],
    )(_gather_kernel)
    return jnp.transpose(kern(item_ids, table_pad))


def kernel(item_ids, embedding_table):
    return _lookup(item_ids.astype(jnp.int32), embedding_table)


# speedup vs baseline: 1.3618x; 1.0301x over previous
import functools

import jax
import jax.numpy as jnp
from jax import lax
from jax.experimental import pallas as pl
from jax.experimental.pallas import tpu as pltpu
from jax.experimental.pallas import tpu_sc as plsc

BATCH = 16384
EMBED_DIM = 64
ROW_PAD = 128

_NUM_WORKERS = 32
_B_PER_W = BATCH // _NUM_WORKERS
_CHUNK = 128
_NCHUNK = _B_PER_W // _CHUNK
_LANES = 16


def _gather_kernel(idx_hbm, table_hbm, out_hbm, idx_v, table_s, g0, g1, g2,
                   g3, t0, t1, gs0, gs1, gs2, gs3, ws0, ws1):
    sid = lax.axis_index("s")
    wid = sid * 2 + lax.axis_index("c")
    base = wid * _B_PER_W

    @pl.when(sid == 0)
    def _():
        pltpu.sync_copy(table_hbm, table_s)

    pltpu.sync_copy(idx_hbm.at[pl.ds(base, _B_PER_W)], idx_v)
    plsc.subcore_barrier()
    g, t = [g0, g1, g2, g3], [t0, t1]
    gs, ws = [gs0, gs1, gs2, gs3], [ws0, ws1]
    gathers = [
        pltpu.async_copy(
            table_s.at[idx_v.at[pl.ds(j * _CHUNK, _CHUNK)]],
            g[j],
            gs[j],
        )
        for j in range(_NCHUNK)
    ]

    iota = lax.iota(jnp.int32, _LANES)
    diags = [jnp.bitwise_and(iota + d, _LANES - 1) for d in range(_LANES)]
    _EBLK = EMBED_DIM // _LANES
    _BBLK = _CHUNK // _LANES
    writes = {}
    for j in range(_NCHUNK):
        gathers[j].wait()
        if j >= 2:
            writes[j - 2].wait()
        gbuf, tbuf = g[j], t[j % 2]

        @plsc.parallel_loop(0, _EBLK * _BBLK, unroll=2)
        def _(blk, gbuf=gbuf, tbuf=tbuf):
            e0 = (blk % _EBLK) * _LANES
            b0 = (blk // _EBLK) * _LANES
            rowv = iota + b0
            for d in range(_LANES):
                colv = diags[d] + e0
                vals = plsc.load_gather(gbuf, [rowv, colv])
                plsc.store_scatter(tbuf, [colv, rowv], vals)

        writes[j] = pltpu.async_copy(
            tbuf,
            out_hbm.at[:, pl.ds(base + j * _CHUNK, _CHUNK)],
            ws[j % 2],
        )
    writes[_NCHUNK - 2].wait()
    writes[_NCHUNK - 1].wait()


@jax.jit
def _lookup(item_ids, embedding_table):
    mesh = plsc.VectorSubcoreMesh(core_axis_name="c", subcore_axis_name="s")
    kern = functools.partial(
        pl.kernel,
        mesh=mesh,
        compiler_params=pltpu.CompilerParams(needs_layout_passes=False),
        out_type=jax.ShapeDtypeStruct((EMBED_DIM, BATCH), jnp.float32),
        scratch_types=[
            pltpu.VMEM((_B_PER_W,), jnp.int32),
            pltpu.VMEM_SHARED((1001, EMBED_DIM), jnp.float32),
            pltpu.VMEM((_CHUNK, EMBED_DIM), jnp.float32),
            pltpu.VMEM((_CHUNK, EMBED_DIM), jnp.float32),
            pltpu.VMEM((_CHUNK, EMBED_DIM), jnp.float32),
            pltpu.VMEM((_CHUNK, EMBED_DIM), jnp.float32),
            pltpu.VMEM((EMBED_DIM, _CHUNK), jnp.float32),
            pltpu.VMEM((EMBED_DIM, _CHUNK), jnp.float32),
            pltpu.SemaphoreType.DMA,
            pltpu.SemaphoreType.DMA,
            pltpu.SemaphoreType.DMA,
            pltpu.SemaphoreType.DMA,
            pltpu.SemaphoreType.DMA,
            pltpu.SemaphoreType.DMA,
        ],
    )(_gather_kernel)
    return jnp.transpose(kern(item_ids, embedding_table))


def kernel(item_ids, embedding_table):
    return _lookup(item_ids.astype(jnp.int32), embedding_table)
